# Initial kernel scaffold; baseline (speedup 1.0000x reference)
import jax
import jax.numpy as jnp
from jax.experimental import pallas as pl


def kernel(node_feature, edge_index, W1a, b1a, W1b, b1b, W2a, b2a, W2b, b2b, W3a, b3a, W3b, b3b):
    raise NotImplementedError("write your pallas kernel here")


# speedup vs baseline: 4.3832x; 4.3832x over previous
import functools

import jax
import jax.numpy as jnp
from jax import lax
from jax.experimental import pallas as pl
from jax.experimental.pallas import tpu as pltpu
from jax.experimental.pallas import tpu_sc as plsc

N = 10000
D = 128
E = 320000
H = 256

NC = 2
NS = 16
NW = NC * NS
EDGES_PER_TILE = E // NW
CHUNK = 80
NCHUNK = EDGES_PER_TILE // CHUNK
N_PAD = 10240
ROWS_PER_TILE = N_PAD // NS
ZCHUNK = 128
NZ = ROWS_PER_TILE // ZCHUNK


def _segsum_sc(x, src, dst):
    mesh = plsc.VectorSubcoreMesh(core_axis_name="c", subcore_axis_name="s")

    @functools.partial(
        pl.kernel,
        out_type=jax.ShapeDtypeStruct((NC, N_PAD, D), jnp.float32),
        mesh=mesh,
        scratch_types=[
            pltpu.VMEM((CHUNK,), jnp.int32),
            pltpu.VMEM((CHUNK,), jnp.int32),
            pltpu.VMEM((CHUNK, D), jnp.float32),
            pltpu.VMEM((ZCHUNK, D), jnp.float32),
            pltpu.VMEM_SHARED((N_PAD, D), jnp.float32),
            pltpu.SemaphoreType.DMA,
        ],
    )
    def k(x_hbm, src_hbm, dst_hbm, out_hbm, src_v, dst_v, rows_v, stage_v, acc, sem):
        c = lax.axis_index("c")
        s = lax.axis_index("s")
        wid = c * NS + s
        row0 = s * ROWS_PER_TILE

        def zrow(i, t):
            def zlane(l, t2):
                stage_v[i, pl.ds(l * 16, 16)] = jnp.zeros((16,), jnp.float32)
                return t2
            return lax.fori_loop(0, D // 16, zlane, t)
        lax.fori_loop(0, ZCHUNK, zrow, 0)

        def zchunk(j, t):
            pltpu.sync_copy(stage_v, acc.at[pl.ds(row0 + j * ZCHUNK, ZCHUNK)])
            return t
        lax.fori_loop(0, NZ, zchunk, 0)
        plsc.subcore_barrier()

        ebase = wid * EDGES_PER_TILE

        def body(i, t):
            off = ebase + i * CHUNK
            pltpu.sync_copy(src_hbm.at[pl.ds(off, CHUNK)], src_v)
            pltpu.sync_copy(dst_hbm.at[pl.ds(off, CHUNK)], dst_v)
            pltpu.async_copy(x_hbm.at[src_v], rows_v, sem).wait()
            pltpu.sync_copy(rows_v, acc.at[dst_v], add=True)
            return t
        lax.fori_loop(0, NCHUNK, body, 0)
        plsc.subcore_barrier()

        def rb(j, t):
            r = row0 + j * ZCHUNK
            pltpu.sync_copy(acc.at[pl.ds(r, ZCHUNK)], stage_v)
            pltpu.sync_copy(stage_v, out_hbm.at[c, pl.ds(r, ZCHUNK)])
            return t
        lax.fori_loop(0, NZ, rb, 0)

    return k(x, src, dst)


ROWBLK = 400
GRID = N // ROWBLK

_rows_spec = pl.BlockSpec((ROWBLK, D), lambda i: (i, 0))
_out_spec = pl.BlockSpec((ROWBLK, D), lambda i: (i, 0))


def _full(shape):
    return pl.BlockSpec(shape, lambda i: tuple(0 for _ in shape))


def _mlp1_tc(hA, hB, x, W1a, b1a, W1b, b1b):
    def body(hA_r, hB_r, x_r, Wa_r, ba_r, Wb_r, bb_r, out_r):
        h = hA_r[...] + hB_r[...]
        z = (jnp.dot(h, Wa_r[0:D, :], preferred_element_type=jnp.float32)
             + jnp.dot(x_r[...], Wa_r[D:2 * D, :], preferred_element_type=jnp.float32)
             + ba_r[...])
        z = jnp.maximum(z, 0.0)
        a = jnp.dot(z, Wb_r[...], preferred_element_type=jnp.float32) + bb_r[...]
        out_r[...] = jnp.maximum(a, 0.0)

    return pl.pallas_call(
        body,
        out_shape=jax.ShapeDtypeStruct((N, D), jnp.float32),
        grid=(GRID,),
        in_specs=[_rows_spec, _rows_spec, _rows_spec,
                  _full((2 * D, H)), _full((1, H)), _full((H, D)), _full((1, D))],
        out_specs=_out_spec,
    )(hA, hB, x, W1a, b1a.reshape(1, H), W1b, b1b.reshape(1, D))


def _mlp2_tc(hA, hB, a1, x, W2a, b2a, W2b, b2b):
    def body(hA_r, hB_r, a1_r, x_r, Wa_r, ba_r, Wb_r, bb_r, out_r):
        h = hA_r[...] + hB_r[...]
        z = (jnp.dot(h, Wa_r[0:D, :], preferred_element_type=jnp.float32)
             + jnp.dot(a1_r[...], Wa_r[D:2 * D, :], preferred_element_type=jnp.float32)
             + jnp.dot(x_r[...], Wa_r[2 * D:3 * D, :], preferred_element_type=jnp.float32)
             + ba_r[...])
        z = jnp.maximum(z, 0.0)
        a = jnp.dot(z, Wb_r[...], preferred_element_type=jnp.float32) + bb_r[...]
        out_r[...] = jnp.maximum(a, 0.0)

    return pl.pallas_call(
        body,
        out_shape=jax.ShapeDtypeStruct((N, D), jnp.float32),
        grid=(GRID,),
        in_specs=[_rows_spec, _rows_spec, _rows_spec, _rows_spec,
                  _full((3 * D, H)), _full((1, H)), _full((H, D)), _full((1, D))],
        out_specs=_out_spec,
    )(hA, hB, a1, x, W2a, b2a.reshape(1, H), W2b, b2b.reshape(1, D))


def _mlp3_tc(hA, hB, a2, x, W3a, b3a, W3b, b3b):
    def body(hA_r, hB_r, a2_r, x_r, Wa_r, ba_r, Wb_r, bb_r, out_r):
        h = hA_r[...] + hB_r[...]
        z = (jnp.dot(h, Wa_r[0:D, :], preferred_element_type=jnp.float32)
             + jnp.dot(a2_r[...], Wa_r[D:2 * D, :], preferred_element_type=jnp.float32)
             + jnp.dot(x_r[...], Wa_r[2 * D:3 * D, :], preferred_element_type=jnp.float32)
             + ba_r[...])
        z = jnp.maximum(z, 0.0)
        logits = jnp.dot(z, Wb_r[...], preferred_element_type=jnp.float32) + bb_r[...]
        m = jnp.max(logits, axis=1, keepdims=True)
        e = jnp.exp(logits - m)
        lse = jnp.log(jnp.sum(e, axis=1, keepdims=True))
        out_r[...] = logits - m - lse

    return pl.pallas_call(
        body,
        out_shape=jax.ShapeDtypeStruct((N, D), jnp.float32),
        grid=(GRID,),
        in_specs=[_rows_spec, _rows_spec, _rows_spec, _rows_spec,
                  _full((3 * D, H)), _full((1, H)), _full((H, D)), _full((1, D))],
        out_specs=_out_spec,
    )(hA, hB, a2, x, W3a, b3a.reshape(1, H), W3b, b3b.reshape(1, D))


def kernel(node_feature, edge_index, W1a, b1a, W1b, b1b,
           W2a, b2a, W2b, b2b, W3a, b3a, W3b, b3b):
    x = node_feature
    src = edge_index[0]
    dst = edge_index[1]

    h1 = _segsum_sc(x, src, dst)
    a1 = _mlp1_tc(h1[0, :N], h1[1, :N], x, W1a, b1a, W1b, b1b)

    h2 = _segsum_sc(a1, src, dst)
    a2 = _mlp2_tc(h2[0, :N], h2[1, :N], a1, x, W2a, b2a, W2b, b2b)

    h3 = _segsum_sc(a2, src, dst)
    return _mlp3_tc(h3[0, :N], h3[1, :N], a2, x, W3a, b3a, W3b, b3b)
